# baseline (speedup 1.0000x reference)
import jax
import jax.numpy as jnp
from jax import lax
from jax.experimental import pallas as pl
from jax.experimental.pallas import tpu as pltpu
from jax.experimental.pallas import tpu_sc as plsc

N_ROWS = 32768
N_COLS = 384
LANES = 16
N_CORES = 2
N_SUBCORES = 16
N_WORKERS = N_CORES * N_SUBCORES
ROWS_PER_WORKER = N_ROWS // N_WORKERS
CHUNK_ROWS = 64
N_CHUNKS = ROWS_PER_WORKER // CHUNK_ROWS
COL_BLOCKS = N_COLS // LANES
NBUF = 2

SUB = 8
LANE = 128
ROW_TILE_WORDS = N_COLS * SUB
CHUNK_WORDS = CHUNK_ROWS * N_COLS
ROW_SPAN = (N_COLS // LANE - 1) * LANE * SUB + LANE


def _permute_body(x_hbm, perm_hbm, out_hbm, *scratch):
    wid = lax.axis_index("s") * N_CORES + lax.axis_index("c")
    base_word = wid * ROWS_PER_WORKER * N_COLS

    bufs_in = scratch[:NBUF]
    bufs_out = scratch[NBUF:2 * NBUF]
    perm_v = scratch[2 * NBUF]
    sems_in = scratch[2 * NBUF + 1:2 * NBUF + 1 + NBUF]
    sems_out = scratch[2 * NBUF + 1 + NBUF:]

    pltpu.sync_copy(perm_hbm, perm_v)
    idx_vecs = []
    for k in range(COL_BLOCKS):
        pv = perm_v[pl.ds(k * LANES, LANES)]
        idx_vecs.append(
            ((pv >> 7) << 10) + (pv & 127))
    out_offs = [(k // 8) * (LANE * SUB) + (k % 8) * LANES
                for k in range(COL_BLOCKS)]

    def in_copy(g, b):
        w0 = base_word + g * CHUNK_WORDS
        return pltpu.make_async_copy(
            x_hbm.at[pl.ds(w0, CHUNK_WORDS)], bufs_in[b], sems_in[b])

    def out_copy(g, b):
        w0 = base_word + g * CHUNK_WORDS
        return pltpu.make_async_copy(
            bufs_out[b], out_hbm.at[pl.ds(w0, CHUNK_WORDS)], sems_out[b])

    def compute_chunk(b):
        @plsc.parallel_loop(0, CHUNK_WORDS, step=ROW_TILE_WORDS)
        def tile_body(tilebase):
            for ri in range(SUB):
                rowbase = tilebase + ri * LANE
                src = bufs_in[b].at[pl.ds(rowbase, ROW_SPAN)]
                dst = bufs_out[b].at[pl.ds(rowbase, ROW_SPAN)]
                for k in range(COL_BLOCKS):
                    dst[pl.ds(out_offs[k], LANES)] = plsc.load_gather(
                        src, [idx_vecs[k]])

    for b in range(NBUF):
        in_copy(b, b).start()

    @pl.loop(0, N_CHUNKS, step=NBUF)
    def chunk_body(g0):
        for b in range(NBUF):
            g = g0 + b

            @pl.when(g >= NBUF)
            def _():
                out_copy(g - NBUF, b).wait()

            in_copy(g, b).wait()
            compute_chunk(b)
            out_copy(g, b).start()

            @pl.when(g < N_CHUNKS - NBUF)
            def _():
                in_copy(g + NBUF, b).start()

    for b in range(NBUF):
        out_copy(N_CHUNKS - NBUF + b, b).wait()


@jax.jit
def kernel(x, perm):
    x_flat = (x.reshape(N_ROWS // SUB, SUB, N_COLS // LANE, LANE)
               .transpose(0, 2, 1, 3).reshape(-1))
    run = pl.kernel(
        _permute_body,
        out_type=jax.ShapeDtypeStruct((N_ROWS * N_COLS,), jnp.float32),
        mesh=plsc.VectorSubcoreMesh(
            core_axis_name="c", subcore_axis_name="s",
            num_cores=N_CORES, num_subcores=N_SUBCORES,
        ),
        scratch_types=(
            [pltpu.VMEM((CHUNK_WORDS,), jnp.float32)] * (2 * NBUF)
            + [pltpu.VMEM((N_COLS,), jnp.int32)]
            + [pltpu.SemaphoreType.DMA] * (2 * NBUF)
        ),
        compiler_params=pltpu.CompilerParams(
            use_tc_tiling_on_sc=False, needs_layout_passes=False),
    )
    out_flat = run(x_flat, perm)
    return (out_flat.reshape(N_ROWS // SUB, N_COLS // LANE, SUB, LANE)
                    .transpose(0, 2, 1, 3).reshape(N_ROWS, N_COLS))


# speedup vs baseline: 1.3996x; 1.3996x over previous
import jax
import jax.numpy as jnp
from jax import lax
from jax.experimental import pallas as pl
from jax.experimental.pallas import tpu as pltpu
from jax.experimental.pallas import tpu_sc as plsc

N_ROWS = 32768
N_COLS = 384
LANES = 16
N_CORES = 2
N_SUBCORES = 16
N_WORKERS = N_CORES * N_SUBCORES
ROWS_PER_WORKER = N_ROWS // N_WORKERS
CHUNK_ROWS = 64
N_CHUNKS = ROWS_PER_WORKER // CHUNK_ROWS
COL_BLOCKS = N_COLS // LANES
NBUF = 2

SUB = 8
LANE = 128
ROW_TILE_WORDS = N_COLS * SUB
CHUNK_WORDS = CHUNK_ROWS * N_COLS
ROW_SPAN = (N_COLS // LANE - 1) * LANE * SUB + LANE


def _permute_body(x_hbm, perm_hbm, out_hbm, *scratch):
    wid = lax.axis_index("s") * N_CORES + lax.axis_index("c")
    base_word = wid * ROWS_PER_WORKER * N_COLS

    bufs_in = scratch[:NBUF]
    bufs_out = scratch[NBUF:2 * NBUF]
    perm_v = scratch[2 * NBUF]
    sems_in = scratch[2 * NBUF + 1:2 * NBUF + 1 + NBUF]
    sems_out = scratch[2 * NBUF + 1 + NBUF:]

    pltpu.sync_copy(perm_hbm, perm_v)
    idx_vecs = []
    for k in range(COL_BLOCKS):
        pv = perm_v[pl.ds(k * LANES, LANES)]
        idx_vecs.append(
            ((pv >> 7) << 10) + (pv & 127))
    out_offs = [(k // 8) * (LANE * SUB) + (k % 8) * LANES
                for k in range(COL_BLOCKS)]

    def in_copy(g, b):
        w0 = base_word + g * CHUNK_WORDS
        return pltpu.make_async_copy(
            x_hbm.at[pl.ds(w0, CHUNK_WORDS)], bufs_in[b], sems_in[b])

    def out_copy(g, b):
        w0 = base_word + g * CHUNK_WORDS
        return pltpu.make_async_copy(
            bufs_out[b], out_hbm.at[pl.ds(w0, CHUNK_WORDS)], sems_out[b])

    def compute_chunk(b):
        @plsc.parallel_loop(0, CHUNK_ROWS, unroll=2)
        def row_body(r):
            rowbase = (r >> 3) * ROW_TILE_WORDS + (r & 7) * LANE
            src = bufs_in[b].at[pl.ds(rowbase, ROW_SPAN)]
            dst = bufs_out[b].at[pl.ds(rowbase, ROW_SPAN)]
            for k in range(COL_BLOCKS):
                dst[pl.ds(out_offs[k], LANES)] = plsc.load_gather(
                    src, [idx_vecs[k]])

    for b in range(NBUF):
        in_copy(b, b).start()

    @pl.loop(0, N_CHUNKS, step=NBUF)
    def chunk_body(g0):
        for b in range(NBUF):
            g = g0 + b

            @pl.when(g >= NBUF)
            def _():
                out_copy(g - NBUF, b).wait()

            in_copy(g, b).wait()
            compute_chunk(b)
            out_copy(g, b).start()

            @pl.when(g < N_CHUNKS - NBUF)
            def _():
                in_copy(g + NBUF, b).start()

    for b in range(NBUF):
        out_copy(N_CHUNKS - NBUF + b, b).wait()


@jax.jit
def kernel(x, perm):
    x_flat = (x.reshape(N_ROWS // SUB, SUB, N_COLS // LANE, LANE)
               .transpose(0, 2, 1, 3).reshape(-1))
    run = pl.kernel(
        _permute_body,
        out_type=jax.ShapeDtypeStruct((N_ROWS * N_COLS,), jnp.float32),
        mesh=plsc.VectorSubcoreMesh(
            core_axis_name="c", subcore_axis_name="s",
            num_cores=N_CORES, num_subcores=N_SUBCORES,
        ),
        scratch_types=(
            [pltpu.VMEM((CHUNK_WORDS,), jnp.float32)] * (2 * NBUF)
            + [pltpu.VMEM((N_COLS,), jnp.int32)]
            + [pltpu.SemaphoreType.DMA] * (2 * NBUF)
        ),
        compiler_params=pltpu.CompilerParams(
            use_tc_tiling_on_sc=False, needs_layout_passes=False),
    )
    out_flat = run(x_flat, perm)
    return (out_flat.reshape(N_ROWS // SUB, N_COLS // LANE, SUB, LANE)
                    .transpose(0, 2, 1, 3).reshape(N_ROWS, N_COLS))


# speedup vs baseline: 1.4008x; 1.0009x over previous
import jax
import jax.numpy as jnp
from jax import lax
from jax.experimental import pallas as pl
from jax.experimental.pallas import tpu as pltpu
from jax.experimental.pallas import tpu_sc as plsc

N_ROWS = 32768
N_COLS = 384
LANES = 16
N_CORES = 2
N_SUBCORES = 16
N_WORKERS = N_CORES * N_SUBCORES
ROWS_PER_WORKER = N_ROWS // N_WORKERS
CHUNK_ROWS = 64
N_CHUNKS = ROWS_PER_WORKER // CHUNK_ROWS
COL_BLOCKS = N_COLS // LANES
NBUF = 2

SUB = 8
LANE = 128
ROW_TILE_WORDS = N_COLS * SUB
CHUNK_WORDS = CHUNK_ROWS * N_COLS
ROW_SPAN = (N_COLS // LANE - 1) * LANE * SUB + LANE


def _permute_body(x_hbm, perm_hbm, out_hbm, *scratch):
    wid = lax.axis_index("s") * N_CORES + lax.axis_index("c")
    base_word = wid * ROWS_PER_WORKER * N_COLS

    bufs_in = scratch[:NBUF]
    bufs_out = scratch[NBUF:2 * NBUF]
    perm_v = scratch[2 * NBUF]
    sems_in = scratch[2 * NBUF + 1:2 * NBUF + 1 + NBUF]
    sems_out = scratch[2 * NBUF + 1 + NBUF:]

    pltpu.sync_copy(perm_hbm, perm_v)
    idx_vecs = []
    for k in range(COL_BLOCKS):
        pv = perm_v[pl.ds(k * LANES, LANES)]
        idx_vecs.append(
            ((pv >> 7) << 10) + (pv & 127))
    out_offs = [(k // 8) * (LANE * SUB) + (k % 8) * LANES
                for k in range(COL_BLOCKS)]

    def in_copy(g, b):
        w0 = base_word + g * CHUNK_WORDS
        return pltpu.make_async_copy(
            x_hbm.at[pl.ds(w0, CHUNK_WORDS)], bufs_in[b], sems_in[b])

    def out_copy(g, b):
        w0 = base_word + g * CHUNK_WORDS
        return pltpu.make_async_copy(
            bufs_out[b], out_hbm.at[pl.ds(w0, CHUNK_WORDS)], sems_out[b])

    def compute_chunk(b):
        @plsc.parallel_loop(0, CHUNK_ROWS, unroll=4)
        def row_body(r):
            rowbase = (r >> 3) * ROW_TILE_WORDS + (r & 7) * LANE
            src = bufs_in[b].at[pl.ds(rowbase, ROW_SPAN)]
            dst = bufs_out[b].at[pl.ds(rowbase, ROW_SPAN)]
            for k in range(COL_BLOCKS):
                dst[pl.ds(out_offs[k], LANES)] = plsc.load_gather(
                    src, [idx_vecs[k]])

    for b in range(NBUF):
        in_copy(b, b).start()

    @pl.loop(0, N_CHUNKS, step=NBUF)
    def chunk_body(g0):
        for b in range(NBUF):
            g = g0 + b

            @pl.when(g >= NBUF)
            def _():
                out_copy(g - NBUF, b).wait()

            in_copy(g, b).wait()
            compute_chunk(b)
            out_copy(g, b).start()

            @pl.when(g < N_CHUNKS - NBUF)
            def _():
                in_copy(g + NBUF, b).start()

    for b in range(NBUF):
        out_copy(N_CHUNKS - NBUF + b, b).wait()


@jax.jit
def kernel(x, perm):
    x_flat = (x.reshape(N_ROWS // SUB, SUB, N_COLS // LANE, LANE)
               .transpose(0, 2, 1, 3).reshape(-1))
    run = pl.kernel(
        _permute_body,
        out_type=jax.ShapeDtypeStruct((N_ROWS * N_COLS,), jnp.float32),
        mesh=plsc.VectorSubcoreMesh(
            core_axis_name="c", subcore_axis_name="s",
            num_cores=N_CORES, num_subcores=N_SUBCORES,
        ),
        scratch_types=(
            [pltpu.VMEM((CHUNK_WORDS,), jnp.float32)] * (2 * NBUF)
            + [pltpu.VMEM((N_COLS,), jnp.int32)]
            + [pltpu.SemaphoreType.DMA] * (2 * NBUF)
        ),
        compiler_params=pltpu.CompilerParams(
            use_tc_tiling_on_sc=False, needs_layout_passes=False),
    )
    out_flat = run(x_flat, perm)
    return (out_flat.reshape(N_ROWS // SUB, N_COLS // LANE, SUB, LANE)
                    .transpose(0, 2, 1, 3).reshape(N_ROWS, N_COLS))


# speedup vs baseline: 1.4224x; 1.0154x over previous
import jax
import jax.numpy as jnp
from jax import lax
from jax.experimental import pallas as pl
from jax.experimental.pallas import tpu as pltpu
from jax.experimental.pallas import tpu_sc as plsc

N_ROWS = 32768
N_COLS = 384
LANES = 16
N_CORES = 2
N_SUBCORES = 16
N_WORKERS = N_CORES * N_SUBCORES
ROWS_PER_WORKER = N_ROWS // N_WORKERS
CHUNK_ROWS = 64
N_CHUNKS = ROWS_PER_WORKER // CHUNK_ROWS
COL_BLOCKS = N_COLS // LANES
NBUF = 2

SUB = 8
LANE = 128
ROW_TILE_WORDS = N_COLS * SUB
CHUNK_WORDS = CHUNK_ROWS * N_COLS
ROW_SPAN = (N_COLS // LANE - 1) * LANE * SUB + LANE


def _permute_body(x_hbm, perm_hbm, out_hbm, *scratch):
    wid = lax.axis_index("s") * N_CORES + lax.axis_index("c")
    base_word = wid * ROWS_PER_WORKER * N_COLS

    bufs_in = scratch[:NBUF]
    bufs_out = scratch[NBUF:2 * NBUF]
    perm_v = scratch[2 * NBUF]
    sems_in = scratch[2 * NBUF + 1:2 * NBUF + 1 + NBUF]
    sems_out = scratch[2 * NBUF + 1 + NBUF:]

    def _prime(b, w0):
        return pltpu.make_async_copy(
            x_hbm.at[pl.ds(w0, CHUNK_WORDS)], bufs_in[b], sems_in[b])
    for b in range(NBUF):
        _prime(b, base_word + b * CHUNK_WORDS).start()

    pltpu.sync_copy(perm_hbm, perm_v)
    idx_vecs = []
    for k in range(COL_BLOCKS):
        pv = perm_v[pl.ds(k * LANES, LANES)]
        idx_vecs.append(
            ((pv >> 7) << 10) + (pv & 127))
    out_offs = [(k // 8) * (LANE * SUB) + (k % 8) * LANES
                for k in range(COL_BLOCKS)]

    def in_copy(g, b):
        w0 = base_word + g * CHUNK_WORDS
        return pltpu.make_async_copy(
            x_hbm.at[pl.ds(w0, CHUNK_WORDS)], bufs_in[b], sems_in[b])

    def out_copy(g, b):
        w0 = base_word + g * CHUNK_WORDS
        return pltpu.make_async_copy(
            bufs_out[b], out_hbm.at[pl.ds(w0, CHUNK_WORDS)], sems_out[b])

    def compute_chunk(b):
        @plsc.parallel_loop(0, CHUNK_ROWS, unroll=4)
        def row_body(r):
            rowbase = (r >> 3) * ROW_TILE_WORDS + (r & 7) * LANE
            src = bufs_in[b].at[pl.ds(rowbase, ROW_SPAN)]
            dst = bufs_out[b].at[pl.ds(rowbase, ROW_SPAN)]
            for k in range(COL_BLOCKS):
                dst[pl.ds(out_offs[k], LANES)] = plsc.load_gather(
                    src, [idx_vecs[k]])

    @pl.loop(0, N_CHUNKS, step=NBUF)
    def chunk_body(g0):
        for b in range(NBUF):
            g = g0 + b

            @pl.when(g >= NBUF)
            def _():
                out_copy(g - NBUF, b).wait()

            in_copy(g, b).wait()
            compute_chunk(b)
            out_copy(g, b).start()

            @pl.when(g < N_CHUNKS - NBUF)
            def _():
                in_copy(g + NBUF, b).start()

    for b in range(NBUF):
        out_copy(N_CHUNKS - NBUF + b, b).wait()


@jax.jit
def kernel(x, perm):
    x_flat = (x.reshape(N_ROWS // SUB, SUB, N_COLS // LANE, LANE)
               .transpose(0, 2, 1, 3).reshape(-1))
    run = pl.kernel(
        _permute_body,
        out_type=jax.ShapeDtypeStruct((N_ROWS * N_COLS,), jnp.float32),
        mesh=plsc.VectorSubcoreMesh(
            core_axis_name="c", subcore_axis_name="s",
            num_cores=N_CORES, num_subcores=N_SUBCORES,
        ),
        scratch_types=(
            [pltpu.VMEM((CHUNK_WORDS,), jnp.float32)] * (2 * NBUF)
            + [pltpu.VMEM((N_COLS,), jnp.int32)]
            + [pltpu.SemaphoreType.DMA] * (2 * NBUF)
        ),
        compiler_params=pltpu.CompilerParams(
            use_tc_tiling_on_sc=False, needs_layout_passes=False),
    )
    out_flat = run(x_flat, perm)
    return (out_flat.reshape(N_ROWS // SUB, N_COLS // LANE, SUB, LANE)
                    .transpose(0, 2, 1, 3).reshape(N_ROWS, N_COLS))
